# baseline (speedup 1.0000x reference)
import functools

import jax
import jax.numpy as jnp
from jax import lax
from jax.experimental import pallas as pl
from jax.experimental.pallas import tpu as pltpu
from jax.experimental.pallas import tpu_sc as plsc

BATCH = 4096
HIST = 200
EMBED_DIM = 64
OUTPUT_DIM = 128

NC = 2
NS = 16
NW = NC * NS

ROWS_PER_W = BATCH // NW
SPLIT = 128
REM = HIST - SPLIT
NBUF = 4


def _sc_pool_body(x_hbm, table_hbm, out_hbm, idx_v, ring_v, out_stage, *sems):
    wid = lax.axis_index("s") * NC + lax.axis_index("c")
    pltpu.sync_copy(x_hbm.at[pl.ds(wid * ROWS_PER_W, ROWS_PER_W)], idx_v)

    def issue(s, r):
        pltpu.make_async_copy(
            table_hbm.at[idx_v.at[r, pl.ds(0, SPLIT)]],
            ring_v.at[s, pl.ds(0, SPLIT)],
            sems[2 * s],
        ).start()
        pltpu.make_async_copy(
            table_hbm.at[idx_v.at[r, pl.ds(SPLIT, REM)]],
            ring_v.at[s, pl.ds(SPLIT, REM)],
            sems[2 * s + 1],
        ).start()

    def wait(s, r):
        pltpu.make_async_copy(
            table_hbm.at[idx_v.at[r, pl.ds(0, SPLIT)]],
            ring_v.at[s, pl.ds(0, SPLIT)],
            sems[2 * s],
        ).wait()
        pltpu.make_async_copy(
            table_hbm.at[idx_v.at[r, pl.ds(SPLIT, REM)]],
            ring_v.at[s, pl.ds(SPLIT, REM)],
            sems[2 * s + 1],
        ).wait()

    for s in range(NBUF):
        issue(s, s)

    scale = jnp.float32(1.0 / HIST)
    z = jnp.zeros((16,), jnp.float32)

    def period(g, carry):
        for j in range(NBUF):
            r = g * NBUF + j
            wait(j, r)

            def body(i, a):
                a0, a1, a2, a3 = a
                a0 = a0 + ring_v[j, i, pl.ds(0, 16)]
                a1 = a1 + ring_v[j, i, pl.ds(16, 16)]
                a2 = a2 + ring_v[j, i, pl.ds(32, 16)]
                a3 = a3 + ring_v[j, i, pl.ds(48, 16)]
                return (a0, a1, a2, a3)

            a0, a1, a2, a3 = lax.fori_loop(0, HIST, body, (z, z, z, z))

            @pl.when(r + NBUF < ROWS_PER_W)
            def _():
                issue(j, r + NBUF)

            out_stage[r, pl.ds(0, 16)] = a0 * scale
            out_stage[r, pl.ds(16, 16)] = a1 * scale
            out_stage[r, pl.ds(32, 16)] = a2 * scale
            out_stage[r, pl.ds(48, 16)] = a3 * scale

        return carry

    lax.fori_loop(0, ROWS_PER_W // NBUF, period, 0)
    pltpu.sync_copy(out_stage, out_hbm.at[pl.ds(wid * ROWS_PER_W, ROWS_PER_W)])


_sc_pool = functools.partial(
    pl.kernel,
    out_type=jax.ShapeDtypeStruct((BATCH, EMBED_DIM), jnp.float32),
    mesh=plsc.VectorSubcoreMesh(core_axis_name="c", subcore_axis_name="s"),
    scratch_types=[
        pltpu.VMEM((ROWS_PER_W, HIST), jnp.int32),
        pltpu.VMEM((NBUF, HIST, EMBED_DIM), jnp.float32),
        pltpu.VMEM((ROWS_PER_W, EMBED_DIM), jnp.float32),
    ]
    + [pltpu.SemaphoreType.DMA] * (2 * NBUF),
    compiler_params=pltpu.CompilerParams(use_tc_tiling_on_sc=False),
)(_sc_pool_body)


def _tc_head_body(p_ref, w_ref, b_ref, o_ref):
    out = jnp.dot(p_ref[...], w_ref[...], preferred_element_type=jnp.float32)
    out = out + b_ref[...]
    ss = jnp.sum(out * out, axis=1, keepdims=True)
    norm = jnp.sqrt(ss)
    o_ref[...] = out / jnp.maximum(norm, 1e-12)


def _tc_head(pooled, wt, b2):
    blk = 512
    return pl.pallas_call(
        _tc_head_body,
        grid=(BATCH // blk,),
        in_specs=[
            pl.BlockSpec((blk, EMBED_DIM), lambda i: (i, 0)),
            pl.BlockSpec((EMBED_DIM, OUTPUT_DIM), lambda i: (0, 0)),
            pl.BlockSpec((1, OUTPUT_DIM), lambda i: (0, 0)),
        ],
        out_specs=pl.BlockSpec((blk, OUTPUT_DIM), lambda i: (i, 0)),
        out_shape=jax.ShapeDtypeStruct((BATCH, OUTPUT_DIM), jnp.float32),
    )(pooled, wt, b2)


def kernel(x, table, W, b):
    pooled = _sc_pool(x, table)
    return _tc_head(pooled, W.T, b.reshape(1, OUTPUT_DIM))
